# Initial kernel scaffold; baseline (speedup 1.0000x reference)
import jax
import jax.numpy as jnp
from jax.experimental import pallas as pl


def kernel(x, edge_index, W1, b1, W2, b2):
    raise NotImplementedError("write your pallas kernel here")


# speedup vs baseline: 18.5815x; 18.5815x over previous
import functools

import jax
import jax.numpy as jnp
from jax import lax
from jax.experimental import pallas as pl
from jax.experimental.pallas import tpu as pltpu
from jax.experimental.pallas import tpu_sc as plsc

N = 10000
E = 320000
D = 128
DH = D // 2

NC = 2
NS = 16
CHUNK = 128

_cpt = -(-E // (NS * CHUNK))
CPT = _cpt + (1 - _cpt % 2)
E_PAD = NS * CPT * CHUNK

RPT = -(-(N + 1) // (NS * 128)) * 128
NPAD = RPT * NS

_MESH = plsc.VectorSubcoreMesh(core_axis_name="c", subcore_axis_name="s")


@functools.partial(
    pl.kernel,
    out_type=jax.ShapeDtypeStruct((NC * NPAD,), jnp.float32),
    mesh=_MESH,
    scratch_types=[
        pltpu.VMEM((CPT, CHUNK), jnp.int32),
        pltpu.VMEM((CHUNK,), jnp.float32),
        pltpu.VMEM_SHARED((NPAD,), jnp.float32),
    ],
)
def _deg_kernel(col_hbm, zeros1_hbm, out_hbm, col_v, ones_v, acc):
    c = lax.axis_index("c")
    s = lax.axis_index("s")
    pltpu.sync_copy(col_hbm.at[s], col_v)
    for i in range(CHUNK // 16):
        ones_v[pl.ds(i * 16, 16)] = jnp.ones((16,), jnp.float32)
    pltpu.sync_copy(zeros1_hbm, acc.at[pl.ds(s * RPT, RPT)])
    plsc.subcore_barrier()

    def body(j, carry):
        pltpu.sync_copy(ones_v, acc.at[col_v.at[j]], add=True)
        return carry

    lax.fori_loop(0, CPT, body, 0)
    plsc.subcore_barrier()
    pltpu.sync_copy(acc.at[pl.ds(s * RPT, RPT)],
                    out_hbm.at[pl.ds(c * NPAD + s * RPT, RPT)])


@functools.partial(
    pl.kernel,
    out_type=jax.ShapeDtypeStruct((NC, NPAD, DH), jnp.float32),
    mesh=_MESH,
    scratch_types=[
        pltpu.VMEM((CPT, CHUNK), jnp.int32),
        pltpu.VMEM((CPT, CHUNK), jnp.int32),
        pltpu.VMEM((CHUNK, DH), jnp.float32),
        pltpu.VMEM((CHUNK, DH), jnp.float32),
        pltpu.VMEM_SHARED((NPAD, DH), jnp.float32),
        pltpu.SemaphoreType.DMA,
        pltpu.SemaphoreType.DMA,
    ],
    compiler_params=pltpu.CompilerParams(use_tc_tiling_on_sc=False),
)
def _msg_kernel(xw_hbm, row_hbm, col_hbm, zeros2_hbm, out_hbm,
                row_v, col_v, buf0, buf1, acc, sem0, sem1):
    c = lax.axis_index("c")
    s = lax.axis_index("s")
    xw_c = xw_hbm.at[c]
    pltpu.sync_copy(row_hbm.at[s], row_v)
    pltpu.sync_copy(col_hbm.at[s], col_v)
    pltpu.sync_copy(zeros2_hbm, acc.at[pl.ds(s * RPT, RPT)])
    plsc.subcore_barrier()

    pltpu.async_copy(xw_c.at[row_v.at[0]], buf0, sem0)

    def pair(i, carry):
        j = 2 * i
        pltpu.make_async_copy(xw_c.at[row_v.at[j]], buf0, sem0).wait()
        pltpu.async_copy(xw_c.at[row_v.at[j + 1]], buf1, sem1)
        pltpu.sync_copy(buf0, acc.at[col_v.at[j]], add=True)
        pltpu.make_async_copy(xw_c.at[row_v.at[j + 1]], buf1, sem1).wait()
        pltpu.async_copy(xw_c.at[row_v.at[j + 2]], buf0, sem0)
        pltpu.sync_copy(buf1, acc.at[col_v.at[j + 1]], add=True)
        return carry

    lax.fori_loop(0, (CPT - 1) // 2, pair, 0)
    pltpu.make_async_copy(xw_c.at[row_v.at[CPT - 1]], buf0, sem0).wait()
    pltpu.sync_copy(buf0, acc.at[col_v.at[CPT - 1]], add=True)

    plsc.subcore_barrier()
    pltpu.sync_copy(acc.at[pl.ds(s * RPT, RPT)],
                    out_hbm.at[c, pl.ds(s * RPT, RPT)])


BN = 2000


def _tc1_body(deg_ref, x_ref, w1_ref, xw1p_ref, dis_ref):
    dis = lax.rsqrt(deg_ref[...] + 1.0)
    xw = jnp.dot(x_ref[...], w1_ref[...], preferred_element_type=jnp.float32)
    xw1p_ref[...] = dis * xw
    dis_ref[...] = dis


_tc1 = pl.pallas_call(
    _tc1_body,
    grid=(N // BN,),
    in_specs=[
        pl.BlockSpec((BN, 1), lambda i: (i, 0)),
        pl.BlockSpec((BN, D), lambda i: (i, 0)),
        pl.BlockSpec((D, D), lambda i: (0, 0)),
    ],
    out_specs=[
        pl.BlockSpec((BN, D), lambda i: (i, 0)),
        pl.BlockSpec((BN, 1), lambda i: (i, 0)),
    ],
    out_shape=[
        jax.ShapeDtypeStruct((N, D), jnp.float32),
        jax.ShapeDtypeStruct((N, 1), jnp.float32),
    ],
)


def _tc2_body(acc_ref, xwp_ref, dis_ref, b_ref, w_ref, out_ref):
    dis = dis_ref[...]
    h = jnp.maximum(dis * (acc_ref[...] + xwp_ref[...]) + b_ref[...], 0.0)
    out_ref[...] = dis * jnp.dot(h, w_ref[...],
                                 preferred_element_type=jnp.float32)


_tc2 = pl.pallas_call(
    _tc2_body,
    grid=(N // BN,),
    in_specs=[
        pl.BlockSpec((BN, D), lambda i: (i, 0)),
        pl.BlockSpec((BN, D), lambda i: (i, 0)),
        pl.BlockSpec((BN, 1), lambda i: (i, 0)),
        pl.BlockSpec((1, D), lambda i: (0, 0)),
        pl.BlockSpec((D, D), lambda i: (0, 0)),
    ],
    out_specs=pl.BlockSpec((BN, D), lambda i: (i, 0)),
    out_shape=jax.ShapeDtypeStruct((N, D), jnp.float32),
)


def _tc3_body(acc_ref, xwp_ref, dis_ref, b_ref, out_ref):
    dis = dis_ref[...]
    out_ref[...] = jnp.maximum(
        dis * (acc_ref[...] + xwp_ref[...]) + b_ref[...], 0.0)


_tc3 = pl.pallas_call(
    _tc3_body,
    grid=(N // BN,),
    in_specs=[
        pl.BlockSpec((BN, D), lambda i: (i, 0)),
        pl.BlockSpec((BN, D), lambda i: (i, 0)),
        pl.BlockSpec((BN, 1), lambda i: (i, 0)),
        pl.BlockSpec((1, D), lambda i: (0, 0)),
    ],
    out_specs=pl.BlockSpec((BN, D), lambda i: (i, 0)),
    out_shape=jax.ShapeDtypeStruct((N, D), jnp.float32),
)


def _message_pass(xwp, row_r, col_r, zeros2):
    xw_split = xwp.reshape(N, NC, DH).transpose(1, 0, 2)
    acc = _msg_kernel(xw_split, row_r, col_r, zeros2)
    return jnp.concatenate([acc[0, :N], acc[1, :N]], axis=1)


def kernel(x, edge_index, W1, b1, W2, b2):
    row = edge_index[0].astype(jnp.int32)
    col = edge_index[1].astype(jnp.int32)
    pad = E_PAD - E
    row_p = jnp.concatenate([row, jnp.zeros((pad,), jnp.int32)])
    col_p = jnp.concatenate([col, jnp.full((pad,), N, jnp.int32)])
    row_r = row_p.reshape(NS, CPT, CHUNK)
    col_r = col_p.reshape(NS, CPT, CHUNK)
    zeros1 = jnp.zeros((RPT,), jnp.float32)
    zeros2 = jnp.zeros((RPT, DH), jnp.float32)

    deg_parts = _deg_kernel(col_r, zeros1)
    deg1 = deg_parts[:N].reshape(N, 1)
    xw1p, dis = _tc1(deg1, x, W1)

    acc1 = _message_pass(xw1p, row_r, col_r, zeros2)
    xw2p = _tc2(acc1, xw1p, dis, b1.reshape(1, D), W2)

    acc2 = _message_pass(xw2p, row_r, col_r, zeros2)
    out = _tc3(acc2, xw2p, dis, b2.reshape(1, D))
    return out
